# baseline (speedup 1.0000x reference)
import functools

import jax
import jax.numpy as jnp
from jax import lax
from jax.experimental import pallas as pl
from jax.experimental.pallas import tpu as pltpu
from jax.experimental.pallas import tpu_sc as plsc

HIDDEN = 128
NC = 2
NS = 16
NW = NC * NS
SUB = 128


@functools.lru_cache(maxsize=None)
def _make(b_pad):
    b_per_w = b_pad // NW
    n_sub = b_per_w // SUB
    mesh = plsc.VectorSubcoreMesh(core_axis_name="c", subcore_axis_name="s")

    @functools.partial(
        pl.kernel,
        mesh=mesh,
        out_type=jax.ShapeDtypeStruct((b_pad, HIDDEN), jnp.float32),
        scratch_types=[
            pltpu.VMEM((b_per_w,), jnp.int32),
            pltpu.VMEM((SUB, HIDDEN), jnp.float32),
            pltpu.SemaphoreType.DMA,
            pltpu.SemaphoreType.DMA,
        ],
    )
    def emb_kernel(idx_hbm, table_hbm, out_hbm, idx_v, rows_v, gsem, ssem):
        wid = lax.axis_index("s") * NC + lax.axis_index("c")
        base = wid * b_per_w
        pltpu.sync_copy(idx_hbm.at[pl.ds(base, b_per_w)], idx_v)

        def body(j, _):
            pltpu.async_copy(
                table_hbm.at[idx_v.at[pl.ds(j * SUB, SUB)]], rows_v, gsem
            ).wait()
            pltpu.async_copy(
                rows_v, out_hbm.at[pl.ds(base + j * SUB, SUB)], ssem
            ).wait()
            return 0

        lax.fori_loop(0, n_sub, body, 0)

    return emb_kernel


def kernel(x_long, emb_weight):
    idx = x_long.reshape(-1).astype(jnp.int32)
    b = idx.shape[0]
    chunk = NW * SUB
    b_pad = ((b + chunk - 1) // chunk) * chunk
    idx_p = jnp.pad(idx, (0, b_pad - b))
    out = _make(b_pad)(idx_p, emb_weight)
    return out[:b]


# speedup vs baseline: 1.5247x; 1.5247x over previous
import functools

import jax
import jax.numpy as jnp
from jax import lax
from jax.experimental import pallas as pl
from jax.experimental.pallas import tpu as pltpu
from jax.experimental.pallas import tpu_sc as plsc

HIDDEN = 128
NC = 2
NS = 16
NW = NC * NS
SUB = 64
NB = 7


@functools.lru_cache(maxsize=None)
def _make(b_pad):
    b_per_w = b_pad // NW
    n_sub = b_per_w // SUB
    n_grp = n_sub // NB
    mesh = plsc.VectorSubcoreMesh(core_axis_name="c", subcore_axis_name="s")

    @functools.partial(
        pl.kernel,
        mesh=mesh,
        out_type=jax.ShapeDtypeStruct((b_pad, HIDDEN), jnp.float32),
        scratch_types=[
            pltpu.VMEM((b_per_w,), jnp.int32),
            pltpu.VMEM((2, NB, SUB, HIDDEN), jnp.float32),
            pltpu.SemaphoreType.DMA,
            pltpu.SemaphoreType.DMA,
        ],
    )
    def emb_kernel(idx_hbm, table_hbm, out_hbm, idx_v, bufs, gsem, ssem):
        wid = lax.axis_index("s") * NC + lax.axis_index("c")
        base = wid * b_per_w
        pltpu.sync_copy(idx_hbm.at[pl.ds(base, b_per_w)], idx_v)

        def fire_gather(chunk, s, b):
            return pltpu.async_copy(
                table_hbm.at[idx_v.at[pl.ds(chunk * SUB, SUB)]],
                bufs.at[s, b],
                gsem,
            )

        def fire_scatter(chunk, s, b):
            return pltpu.async_copy(
                bufs.at[s, b],
                out_hbm.at[pl.ds(base + chunk * SUB, SUB)],
                ssem,
            )

        gh = [fire_gather(b, 0, b) for b in range(NB)]
        sh = []
        for g in range(n_grp):
            s = g % 2
            for h in gh:
                h.wait()
            for h in sh:
                h.wait()
            if g + 1 < n_grp:
                gh = [
                    fire_gather((g + 1) * NB + b, 1 - s, b) for b in range(NB)
                ]
            else:
                gh = []
            sh = [fire_scatter(g * NB + b, s, b) for b in range(NB)]
        for h in sh:
            h.wait()

    return emb_kernel


def kernel(x_long, emb_weight):
    idx = x_long.reshape(-1).astype(jnp.int32)
    b = idx.shape[0]
    chunk = NW * SUB * NB
    b_pad = ((b + chunk - 1) // chunk) * chunk
    idx_p = jnp.pad(idx, (0, b_pad - b))
    out = _make(b_pad)(idx_p, emb_weight)
    return out[:b]


# speedup vs baseline: 1.5284x; 1.0024x over previous
import functools

import jax
import jax.numpy as jnp
from jax import lax
from jax.experimental import pallas as pl
from jax.experimental.pallas import tpu as pltpu
from jax.experimental.pallas import tpu_sc as plsc

HIDDEN = 128
NC = 2
NS = 16
NW = NC * NS
SUB = 224
NBUF = 4
LOOKAHEAD = 2


@functools.lru_cache(maxsize=None)
def _make(b_pad):
    b_per_w = b_pad // NW
    n_sub = b_per_w // SUB
    mesh = plsc.VectorSubcoreMesh(core_axis_name="c", subcore_axis_name="s")

    @functools.partial(
        pl.kernel,
        mesh=mesh,
        out_type=jax.ShapeDtypeStruct((b_pad, HIDDEN), jnp.float32),
        scratch_types=[
            pltpu.VMEM((b_per_w,), jnp.int32),
            pltpu.VMEM((NBUF, SUB, HIDDEN), jnp.float32),
            pltpu.SemaphoreType.DMA,
            pltpu.SemaphoreType.DMA,
        ],
    )
    def emb_kernel(idx_hbm, table_hbm, out_hbm, idx_v, bufs, gsem, ssem):
        wid = lax.axis_index("s") * NC + lax.axis_index("c")
        base = wid * b_per_w
        pltpu.sync_copy(idx_hbm.at[pl.ds(base, b_per_w)], idx_v)

        def fire_gather(chunk):
            return pltpu.async_copy(
                table_hbm.at[idx_v.at[pl.ds(chunk * SUB, SUB)]],
                bufs.at[chunk % NBUF],
                gsem,
            )

        def fire_scatter(chunk):
            return pltpu.async_copy(
                bufs.at[chunk % NBUF],
                out_hbm.at[pl.ds(base + chunk * SUB, SUB)],
                ssem,
            )

        gh = {j: fire_gather(j) for j in range(min(LOOKAHEAD, n_sub))}
        sh = {}
        sdone = 0
        for j in range(n_sub):
            gh[j].wait()
            sh[j] = fire_scatter(j)
            jj = j + LOOKAHEAD
            if jj < n_sub:
                while sdone <= jj - NBUF:
                    sh[sdone].wait()
                    sdone += 1
                gh[jj] = fire_gather(jj)
        while sdone < n_sub:
            sh[sdone].wait()
            sdone += 1

    return emb_kernel


def kernel(x_long, emb_weight):
    idx = x_long.reshape(-1).astype(jnp.int32)
    b = idx.shape[0]
    chunk = NW * SUB
    b_pad = ((b + chunk - 1) // chunk) * chunk
    idx_p = jnp.pad(idx, (0, b_pad - b))
    out = _make(b_pad)(idx_p, emb_weight)
    return out[:b]


# speedup vs baseline: 3.8647x; 2.5286x over previous
import functools

import jax
import jax.numpy as jnp
from jax import lax
from jax.experimental import pallas as pl
from jax.experimental.pallas import tpu as pltpu
from jax.experimental.pallas import tpu_sc as plsc

HIDDEN = 128
NC = 2
NS = 16
NW = NC * NS
SUB = 224
NBUF = 4
LOOKAHEAD = 2


@functools.lru_cache(maxsize=None)
def _make(b_pad):
    b_per_w = b_pad // NW
    n_sub = b_per_w // SUB
    mesh = plsc.VectorSubcoreMesh(core_axis_name="c", subcore_axis_name="s")

    @functools.partial(
        pl.kernel,
        mesh=mesh,
        out_type=jax.ShapeDtypeStruct((b_pad, HIDDEN), jnp.float32),
        scratch_types=[
            pltpu.VMEM((b_per_w,), jnp.int32),
            pltpu.VMEM((NBUF, SUB, HIDDEN), jnp.float32),
            pltpu.VMEM_SHARED((128, HIDDEN), jnp.float32),
            pltpu.SemaphoreType.DMA,
            pltpu.SemaphoreType.DMA,
        ],
    )
    def emb_kernel(idx_hbm, table_hbm, out_hbm, idx_v, bufs, tbl_sh, gsem, ssem):
        wid = lax.axis_index("s") * NC + lax.axis_index("c")
        base = wid * b_per_w
        sid = lax.axis_index("s")

        @pl.when(sid == 0)
        def _():
            pltpu.sync_copy(table_hbm, tbl_sh)

        pltpu.sync_copy(idx_hbm.at[pl.ds(base, b_per_w)], idx_v)
        plsc.subcore_barrier()

        def fire_gather(chunk):
            return pltpu.async_copy(
                tbl_sh.at[idx_v.at[pl.ds(chunk * SUB, SUB)]],
                bufs.at[chunk % NBUF],
                gsem,
            )

        def fire_scatter(chunk):
            return pltpu.async_copy(
                bufs.at[chunk % NBUF],
                out_hbm.at[pl.ds(base + chunk * SUB, SUB)],
                ssem,
            )

        gh = {j: fire_gather(j) for j in range(min(LOOKAHEAD, n_sub))}
        sh = {}
        sdone = 0
        for j in range(n_sub):
            gh[j].wait()
            sh[j] = fire_scatter(j)
            jj = j + LOOKAHEAD
            if jj < n_sub:
                while sdone <= jj - NBUF:
                    sh[sdone].wait()
                    sdone += 1
                gh[jj] = fire_gather(jj)
        while sdone < n_sub:
            sh[sdone].wait()
            sdone += 1

    return emb_kernel


def kernel(x_long, emb_weight):
    idx = x_long.reshape(-1).astype(jnp.int32)
    b = idx.shape[0]
    chunk = NW * SUB
    b_pad = ((b + chunk - 1) // chunk) * chunk
    idx_p = jnp.pad(idx, (0, b_pad - b))
    out = _make(b_pad)(idx_p, emb_weight)
    return out[:b]
